# Initial kernel scaffold; baseline (speedup 1.0000x reference)
import functools

import jax
import jax.numpy as jnp
from jax import lax
from jax.experimental import pallas as pl
from jax.experimental.pallas import tpu as pltpu

N = 10000
E = 320000
E_LG = 640000
ND = 128
ED = 128

_EPS = 1e-5


def _k1_body(lgx_ref, w12t_ref, bl_ref, a_ref, b_ref):
    ab = jnp.dot(lgx_ref[...], w12t_ref[...], preferred_element_type=jnp.float32)
    a_ref[...] = ab[:, :ED]
    b_ref[...] = ab[:, ED:] + bl_ref[...]


def _k1(lg_x, w12t, b_line):
    blk = 512
    grid = E // blk
    return pl.pallas_call(
        _k1_body,
        grid=(grid,),
        in_specs=[
            pl.BlockSpec((blk, ED), lambda i: (i, 0)),
            pl.BlockSpec((ED, 2 * ED), lambda i: (0, 0)),
            pl.BlockSpec((1, ED), lambda i: (0, 0)),
        ],
        out_specs=[
            pl.BlockSpec((blk, ED), lambda i: (i, 0)),
            pl.BlockSpec((blk, ED), lambda i: (i, 0)),
        ],
        out_shape=[
            jax.ShapeDtypeStruct((E, ED), jnp.float32),
            jax.ShapeDtypeStruct((E, ED), jnp.float32),
        ],
    )(lg_x, w12t, b_line.reshape(1, ED))


def _k3_body(ea_ref, lgx_ref, wgt_ref, bg_ref, g_ref, bt_ref, out_ea_ref, gated_ref):
    s = ea_ref[...] + lgx_ref[...]
    m = jnp.mean(s, axis=1, keepdims=True)
    v = jnp.mean((s - m) ** 2, axis=1, keepdims=True)
    ea = (s - m) / jnp.sqrt(v + _EPS) * g_ref[...] + bt_ref[...]
    gate = jax.nn.sigmoid(
        jnp.dot(ea, wgt_ref[...], preferred_element_type=jnp.float32) + bg_ref[...])
    out_ea_ref[...] = ea
    gated_ref[...] = gate * ea


def _k3(edge_attr, lg_x_new, wgt, b_gate, g_edge, bt_edge):
    blk = 512
    grid = E // blk
    return pl.pallas_call(
        _k3_body,
        grid=(grid,),
        in_specs=[
            pl.BlockSpec((blk, ED), lambda i: (i, 0)),
            pl.BlockSpec((blk, ED), lambda i: (i, 0)),
            pl.BlockSpec((ED, ED), lambda i: (0, 0)),
            pl.BlockSpec((1, ED), lambda i: (0, 0)),
            pl.BlockSpec((1, ED), lambda i: (0, 0)),
            pl.BlockSpec((1, ED), lambda i: (0, 0)),
        ],
        out_specs=[
            pl.BlockSpec((blk, ED), lambda i: (i, 0)),
            pl.BlockSpec((blk, ED), lambda i: (i, 0)),
        ],
        out_shape=[
            jax.ShapeDtypeStruct((E, ED), jnp.float32),
            jax.ShapeDtypeStruct((E, ED), jnp.float32),
        ],
    )(edge_attr, lg_x_new, wgt, b_gate.reshape(1, ED), g_edge.reshape(1, ED),
      bt_edge.reshape(1, ED))


def _k5_body(x_ref, aggp_ref, wat_ref, ba_ref, g_ref, bt_ref, out_ref):
    x = x_ref[...]
    agg = aggp_ref[0] + aggp_ref[1]
    z = (jnp.dot(x, wat_ref[..., :ND], preferred_element_type=jnp.float32)
         + jnp.dot(agg, wat_ref[..., ND:], preferred_element_type=jnp.float32)
         + ba_ref[...])
    h = z * jax.nn.sigmoid(z)
    s = x + h
    m = jnp.mean(s, axis=1, keepdims=True)
    v = jnp.mean((s - m) ** 2, axis=1, keepdims=True)
    out_ref[...] = (s - m) / jnp.sqrt(v + _EPS) * g_ref[...] + bt_ref[...]


def _k5(x, agg_parts, wat, b_atom, g_node, bt_node):
    blk = 1000
    grid = N // blk
    return pl.pallas_call(
        _k5_body,
        grid=(grid,),
        in_specs=[
            pl.BlockSpec((blk, ND), lambda i: (i, 0)),
            pl.BlockSpec((2, blk, ND), lambda i: (0, i, 0)),
            pl.BlockSpec((ND, ND + ED), lambda i: (0, 0)),
            pl.BlockSpec((1, ND), lambda i: (0, 0)),
            pl.BlockSpec((1, ND), lambda i: (0, 0)),
            pl.BlockSpec((1, ND), lambda i: (0, 0)),
        ],
        out_specs=pl.BlockSpec((blk, ND), lambda i: (i, 0)),
        out_shape=jax.ShapeDtypeStruct((N, ND), jnp.float32),
    )(x, agg_parts, wat, b_atom.reshape(1, ND), g_node.reshape(1, ND),
      bt_node.reshape(1, ND))


def kernel(x, edge_index, edge_attr, lg_x, lg_edge_index, lg_edge_attr,
           W_line, b_line, W_gate, b_gate, W_atom, b_atom,
           g_node, bt_node, g_edge, bt_edge):
    lg_src = lg_edge_index[0].astype(jnp.int32)
    lg_dst = lg_edge_index[1].astype(jnp.int32)
    col = edge_index[1].astype(jnp.int32)
    lga = lg_edge_attr[:, 0]
    w3 = W_line[:, 2 * ED]

    w12t = W_line[:, :2 * ED]
    a_rows, b_rows = _k1(lg_x, w12t, b_line)

    msg = a_rows[lg_src] + b_rows[lg_dst] + lga[:, None] * w3[None, :]
    msg = msg * jax.nn.sigmoid(msg)
    sums = jax.ops.segment_sum(msg, lg_dst, num_segments=E)
    cnt = jax.ops.segment_sum(jnp.ones((E_LG,), jnp.float32), lg_dst, num_segments=E)
    lg_x_new = lg_x + sums / jnp.maximum(cnt, 1.0)[:, None]

    ea_new, gated = _k3(edge_attr, lg_x_new, W_gate.T, b_gate, g_edge, bt_edge)

    agg = jax.ops.segment_sum(gated, col, num_segments=N)
    agg_parts = jnp.stack([agg, jnp.zeros_like(agg)])

    x_out = _k5(x, agg_parts, W_atom.T, b_atom, g_node, bt_node)
    return (x_out, ea_new, lg_x_new)


# speedup vs baseline: 4.6266x; 4.6266x over previous
import functools

import jax
import jax.numpy as jnp
from jax import lax
from jax.experimental import pallas as pl
from jax.experimental.pallas import tpu as pltpu

N = 10000
E = 320000
E_LG = 640000
ND = 128
ED = 128

_EPS = 1e-5


def _k1_body(lgx_ref, w12t_ref, bl_ref, a_ref, b_ref):
    ab = jnp.dot(lgx_ref[...], w12t_ref[...], preferred_element_type=jnp.float32)
    a_ref[...] = ab[:, :ED]
    b_ref[...] = ab[:, ED:] + bl_ref[...]


def _k1(lg_x, w12t, b_line):
    blk = 512
    grid = E // blk
    return pl.pallas_call(
        _k1_body,
        grid=(grid,),
        in_specs=[
            pl.BlockSpec((blk, ED), lambda i: (i, 0)),
            pl.BlockSpec((ED, 2 * ED), lambda i: (0, 0)),
            pl.BlockSpec((1, ED), lambda i: (0, 0)),
        ],
        out_specs=[
            pl.BlockSpec((blk, ED), lambda i: (i, 0)),
            pl.BlockSpec((blk, ED), lambda i: (i, 0)),
        ],
        out_shape=[
            jax.ShapeDtypeStruct((E, ED), jnp.float32),
            jax.ShapeDtypeStruct((E, ED), jnp.float32),
        ],
    )(lg_x, w12t, b_line.reshape(1, ED))


def _k3_body(ea_ref, lgx_ref, wgt_ref, bg_ref, g_ref, bt_ref, out_ea_ref, gated_ref):
    s = ea_ref[...] + lgx_ref[...]
    m = jnp.mean(s, axis=1, keepdims=True)
    v = jnp.mean((s - m) ** 2, axis=1, keepdims=True)
    ea = (s - m) / jnp.sqrt(v + _EPS) * g_ref[...] + bt_ref[...]
    gate = jax.nn.sigmoid(
        jnp.dot(ea, wgt_ref[...], preferred_element_type=jnp.float32) + bg_ref[...])
    out_ea_ref[...] = ea
    gated_ref[...] = gate * ea


def _k3(edge_attr, lg_x_new, wgt, b_gate, g_edge, bt_edge):
    blk = 512
    grid = E // blk
    return pl.pallas_call(
        _k3_body,
        grid=(grid,),
        in_specs=[
            pl.BlockSpec((blk, ED), lambda i: (i, 0)),
            pl.BlockSpec((blk, ED), lambda i: (i, 0)),
            pl.BlockSpec((ED, ED), lambda i: (0, 0)),
            pl.BlockSpec((1, ED), lambda i: (0, 0)),
            pl.BlockSpec((1, ED), lambda i: (0, 0)),
            pl.BlockSpec((1, ED), lambda i: (0, 0)),
        ],
        out_specs=[
            pl.BlockSpec((blk, ED), lambda i: (i, 0)),
            pl.BlockSpec((blk, ED), lambda i: (i, 0)),
        ],
        out_shape=[
            jax.ShapeDtypeStruct((E, ED), jnp.float32),
            jax.ShapeDtypeStruct((E, ED), jnp.float32),
        ],
    )(edge_attr, lg_x_new, wgt, b_gate.reshape(1, ED), g_edge.reshape(1, ED),
      bt_edge.reshape(1, ED))


def _k5_body(x_ref, aggp_ref, wat_ref, ba_ref, g_ref, bt_ref, out_ref):
    x = x_ref[...]
    agg = aggp_ref[0] + aggp_ref[1]
    z = (jnp.dot(x, wat_ref[:ND, :], preferred_element_type=jnp.float32)
         + jnp.dot(agg, wat_ref[ND:, :], preferred_element_type=jnp.float32)
         + ba_ref[...])
    h = z * jax.nn.sigmoid(z)
    s = x + h
    m = jnp.mean(s, axis=1, keepdims=True)
    v = jnp.mean((s - m) ** 2, axis=1, keepdims=True)
    out_ref[...] = (s - m) / jnp.sqrt(v + _EPS) * g_ref[...] + bt_ref[...]


def _k5(x, agg_parts, wat, b_atom, g_node, bt_node):
    blk = 1000
    grid = N // blk
    return pl.pallas_call(
        _k5_body,
        grid=(grid,),
        in_specs=[
            pl.BlockSpec((blk, ND), lambda i: (i, 0)),
            pl.BlockSpec((2, blk, ND), lambda i: (0, i, 0)),
            pl.BlockSpec((ND + ED, ND), lambda i: (0, 0)),
            pl.BlockSpec((1, ND), lambda i: (0, 0)),
            pl.BlockSpec((1, ND), lambda i: (0, 0)),
            pl.BlockSpec((1, ND), lambda i: (0, 0)),
        ],
        out_specs=pl.BlockSpec((blk, ND), lambda i: (i, 0)),
        out_shape=jax.ShapeDtypeStruct((N, ND), jnp.float32),
    )(x, agg_parts, wat, b_atom.reshape(1, ND), g_node.reshape(1, ND),
      bt_node.reshape(1, ND))


def kernel(x, edge_index, edge_attr, lg_x, lg_edge_index, lg_edge_attr,
           W_line, b_line, W_gate, b_gate, W_atom, b_atom,
           g_node, bt_node, g_edge, bt_edge):
    lg_src = lg_edge_index[0].astype(jnp.int32)
    lg_dst = lg_edge_index[1].astype(jnp.int32)
    col = edge_index[1].astype(jnp.int32)
    lga = lg_edge_attr[:, 0]
    w3 = W_line[:, 2 * ED]

    w12t = jnp.concatenate([W_line[:, :ED].T, W_line[:, ED:2 * ED].T], axis=1)
    a_rows, b_rows = _k1(lg_x, w12t, b_line)

    msg = a_rows[lg_src] + b_rows[lg_dst] + lga[:, None] * w3[None, :]
    msg = msg * jax.nn.sigmoid(msg)
    sums = jax.ops.segment_sum(msg, lg_dst, num_segments=E)
    cnt = jax.ops.segment_sum(jnp.ones((E_LG,), jnp.float32), lg_dst, num_segments=E)
    lg_x_new = lg_x + sums / jnp.maximum(cnt, 1.0)[:, None]

    ea_new, gated = _k3(edge_attr, lg_x_new, W_gate.T, b_gate, g_edge, bt_edge)

    agg = jax.ops.segment_sum(gated, col, num_segments=N)
    agg_parts = jnp.stack([agg, jnp.zeros_like(agg)])

    x_out = _k5(x, agg_parts, W_atom.T, b_atom, g_node, bt_node)
    return (x_out, ea_new, lg_x_new)


# speedup vs baseline: 5.0565x; 1.0929x over previous
import functools

import jax
import jax.numpy as jnp
from jax import lax
from jax.experimental import pallas as pl
from jax.experimental.pallas import tpu as pltpu
from jax.experimental.pallas import tpu_sc as plsc

N = 10000
E = 320000
E_LG = 640000
ND = 128
ED = 128

_EPS = 1e-5


def _k1_body(lgx_ref, w12t_ref, bl_ref, a_ref, b_ref):
    ab = jnp.dot(lgx_ref[...], w12t_ref[...], preferred_element_type=jnp.float32)
    a_ref[...] = ab[:, :ED]
    b_ref[...] = ab[:, ED:] + bl_ref[...]


def _k1(lg_x, w12t, b_line):
    blk = 512
    grid = E // blk
    return pl.pallas_call(
        _k1_body,
        grid=(grid,),
        in_specs=[
            pl.BlockSpec((blk, ED), lambda i: (i, 0)),
            pl.BlockSpec((ED, 2 * ED), lambda i: (0, 0)),
            pl.BlockSpec((1, ED), lambda i: (0, 0)),
        ],
        out_specs=[
            pl.BlockSpec((blk, ED), lambda i: (i, 0)),
            pl.BlockSpec((blk, ED), lambda i: (i, 0)),
        ],
        out_shape=[
            jax.ShapeDtypeStruct((E, ED), jnp.float32),
            jax.ShapeDtypeStruct((E, ED), jnp.float32),
        ],
    )(lg_x, w12t, b_line.reshape(1, ED))


def _k3_body(ea_ref, lgx_ref, wgt_ref, bg_ref, g_ref, bt_ref, out_ea_ref, gated_ref):
    s = ea_ref[...] + lgx_ref[...]
    m = jnp.mean(s, axis=1, keepdims=True)
    v = jnp.mean((s - m) ** 2, axis=1, keepdims=True)
    ea = (s - m) / jnp.sqrt(v + _EPS) * g_ref[...] + bt_ref[...]
    gate = jax.nn.sigmoid(
        jnp.dot(ea, wgt_ref[...], preferred_element_type=jnp.float32) + bg_ref[...])
    out_ea_ref[...] = ea
    gated_ref[...] = gate * ea


def _k3(edge_attr, lg_x_new, wgt, b_gate, g_edge, bt_edge):
    blk = 512
    grid = E // blk
    return pl.pallas_call(
        _k3_body,
        grid=(grid,),
        in_specs=[
            pl.BlockSpec((blk, ED), lambda i: (i, 0)),
            pl.BlockSpec((blk, ED), lambda i: (i, 0)),
            pl.BlockSpec((ED, ED), lambda i: (0, 0)),
            pl.BlockSpec((1, ED), lambda i: (0, 0)),
            pl.BlockSpec((1, ED), lambda i: (0, 0)),
            pl.BlockSpec((1, ED), lambda i: (0, 0)),
        ],
        out_specs=[
            pl.BlockSpec((blk, ED), lambda i: (i, 0)),
            pl.BlockSpec((blk, ED), lambda i: (i, 0)),
        ],
        out_shape=[
            jax.ShapeDtypeStruct((E, ED), jnp.float32),
            jax.ShapeDtypeStruct((E, ED), jnp.float32),
        ],
    )(edge_attr, lg_x_new, wgt, b_gate.reshape(1, ED), g_edge.reshape(1, ED),
      bt_edge.reshape(1, ED))


_SC_CORES = 2
_SC_TILES = 16
_SC_WORKERS = _SC_CORES * _SC_TILES
_K4_CH = 128


_K4_ZR = 40


def _k4_body(gated_hbm, col_hbm, zeros_hbm, out_hbm, idx_v, rows_v, acc_sh):
    c = lax.axis_index("c")
    s = lax.axis_index("s")
    w = s * _SC_CORES + c
    nzch = N // _K4_ZR

    def zbody(i, carry):
        r = (s + i * _SC_TILES) * _K4_ZR
        pltpu.sync_copy(zeros_hbm, acc_sh.at[pl.ds(r, _K4_ZR)])
        return carry

    nz = (nzch - s + _SC_TILES - 1) // _SC_TILES
    lax.fori_loop(0, nz, zbody, 0)
    plsc.subcore_barrier()

    nch = E // _K4_CH
    nmine = (nch - w + _SC_WORKERS - 1) // _SC_WORKERS

    def body(i, carry):
        off = (w + i * _SC_WORKERS) * _K4_CH
        pltpu.sync_copy(col_hbm.at[pl.ds(off, _K4_CH)], idx_v)
        pltpu.sync_copy(gated_hbm.at[pl.ds(off, _K4_CH), :], rows_v)
        pltpu.sync_copy(rows_v, acc_sh.at[idx_v], add=True)
        return carry

    lax.fori_loop(0, nmine, body, 0)
    plsc.subcore_barrier()

    def fbody(i, carry):
        r = (s + i * _SC_TILES) * _K4_ZR
        pltpu.sync_copy(acc_sh.at[pl.ds(r, _K4_ZR)], out_hbm.at[c, pl.ds(r, _K4_ZR)])
        return carry

    lax.fori_loop(0, nz, fbody, 0)


def _k4(gated, col):
    mesh = plsc.VectorSubcoreMesh(core_axis_name="c", subcore_axis_name="s")
    kfn = pl.kernel(
        _k4_body,
        out_type=jax.ShapeDtypeStruct((_SC_CORES, N, ND), jnp.float32),
        mesh=mesh,
        scratch_types=[
            pltpu.VMEM((_K4_CH,), jnp.int32),
            pltpu.VMEM((_K4_CH, ND), jnp.float32),
            pltpu.VMEM_SHARED((N, ND), jnp.float32),
        ],
    )
    zeros = jnp.zeros((_K4_ZR, ND), jnp.float32)
    return kfn(gated, col, zeros)


def _k5_body(x_ref, aggp_ref, wat_ref, ba_ref, g_ref, bt_ref, out_ref):
    x = x_ref[...]
    agg = aggp_ref[0] + aggp_ref[1]
    z = (jnp.dot(x, wat_ref[:ND, :], preferred_element_type=jnp.float32)
         + jnp.dot(agg, wat_ref[ND:, :], preferred_element_type=jnp.float32)
         + ba_ref[...])
    h = z * jax.nn.sigmoid(z)
    s = x + h
    m = jnp.mean(s, axis=1, keepdims=True)
    v = jnp.mean((s - m) ** 2, axis=1, keepdims=True)
    out_ref[...] = (s - m) / jnp.sqrt(v + _EPS) * g_ref[...] + bt_ref[...]


def _k5(x, agg_parts, wat, b_atom, g_node, bt_node):
    blk = 1000
    grid = N // blk
    return pl.pallas_call(
        _k5_body,
        grid=(grid,),
        in_specs=[
            pl.BlockSpec((blk, ND), lambda i: (i, 0)),
            pl.BlockSpec((2, blk, ND), lambda i: (0, i, 0)),
            pl.BlockSpec((ND + ED, ND), lambda i: (0, 0)),
            pl.BlockSpec((1, ND), lambda i: (0, 0)),
            pl.BlockSpec((1, ND), lambda i: (0, 0)),
            pl.BlockSpec((1, ND), lambda i: (0, 0)),
        ],
        out_specs=pl.BlockSpec((blk, ND), lambda i: (i, 0)),
        out_shape=jax.ShapeDtypeStruct((N, ND), jnp.float32),
    )(x, agg_parts, wat, b_atom.reshape(1, ND), g_node.reshape(1, ND),
      bt_node.reshape(1, ND))


def kernel(x, edge_index, edge_attr, lg_x, lg_edge_index, lg_edge_attr,
           W_line, b_line, W_gate, b_gate, W_atom, b_atom,
           g_node, bt_node, g_edge, bt_edge):
    lg_src = lg_edge_index[0].astype(jnp.int32)
    lg_dst = lg_edge_index[1].astype(jnp.int32)
    col = edge_index[1].astype(jnp.int32)
    lga = lg_edge_attr[:, 0]
    w3 = W_line[:, 2 * ED]

    w12t = jnp.concatenate([W_line[:, :ED].T, W_line[:, ED:2 * ED].T], axis=1)
    a_rows, b_rows = _k1(lg_x, w12t, b_line)

    msg = a_rows[lg_src] + b_rows[lg_dst] + lga[:, None] * w3[None, :]
    msg = msg * jax.nn.sigmoid(msg)
    sums = jax.ops.segment_sum(msg, lg_dst, num_segments=E)
    cnt = jax.ops.segment_sum(jnp.ones((E_LG,), jnp.float32), lg_dst, num_segments=E)
    lg_x_new = lg_x + sums / jnp.maximum(cnt, 1.0)[:, None]

    ea_new, gated = _k3(edge_attr, lg_x_new, W_gate.T, b_gate, g_edge, bt_edge)

    agg_parts = _k4(gated, col)

    x_out = _k5(x, agg_parts, W_atom.T, b_atom, g_node, bt_node)
    return (x_out, ea_new, lg_x_new)
